# Initial kernel scaffold; baseline (speedup 1.0000x reference)
import jax
import jax.numpy as jnp
from jax.experimental import pallas as pl


def kernel(x, edge_index, W1, b1, g1, be1, W2, b2, g2, be2, Wc, bc):
    raise NotImplementedError("write your pallas kernel here")


# speedup vs baseline: 6.6359x; 6.6359x over previous
import functools

import jax
import jax.numpy as jnp
from jax import lax
from jax.experimental import pallas as pl
from jax.experimental.pallas import tpu as pltpu
from jax.experimental.pallas import tpu_sc as plsc

N_NODES = 10000
N_PAD = 10240
N_EDGES = 320000
E_PAD = 327680
F_DIM = 128
OUT_DIM = 64
EPS = 1e-5

SC_CORES = 2
SC_SUBCORES = 16
N_WORKERS = SC_CORES * SC_SUBCORES

ECHUNK = 128
EROWS_W = 80
SUB_ROWS = N_PAD // SC_SUBCORES

BM = 1000
GRID = N_NODES // BM


@functools.cache
def _sc_mesh():
    return plsc.VectorSubcoreMesh(
        core_axis_name="c", subcore_axis_name="s",
        num_cores=SC_CORES, num_subcores=SC_SUBCORES)


def _fill_const(ref, nrows, ncols, value):
    @pl.loop(0, nrows)
    def _(r):
        @pl.loop(0, ncols // 16)
        def _(c):
            ref[r, pl.ds(c * 16, 16)] = jnp.full((16,), value, jnp.float32)


def _sc_degree(dst3):
    @functools.partial(
        pl.kernel,
        out_type=jax.ShapeDtypeStruct((SC_CORES, N_PAD, F_DIM), jnp.float32),
        mesh=_sc_mesh(),
        scratch_types=[
            pltpu.VMEM((EROWS_W, ECHUNK), jnp.int32),
            pltpu.VMEM((ECHUNK, F_DIM), jnp.float32),
            pltpu.VMEM_SHARED((N_PAD, F_DIM), jnp.float32),
        ],
    )
    def k(dst_hbm, out_hbm, dst_v, ones_v, acc_sh):
        cid = lax.axis_index("c")
        sid = lax.axis_index("s")
        wid = sid * SC_CORES + cid

        _fill_const(ones_v, ECHUNK, F_DIM, 0.0)

        @pl.loop(0, SUB_ROWS // ECHUNK)
        def _(q):
            pltpu.sync_copy(
                ones_v,
                acc_sh.at[pl.ds(sid * SUB_ROWS + q * ECHUNK, ECHUNK)])

        _fill_const(ones_v, ECHUNK, F_DIM, 1.0)
        pltpu.sync_copy(dst_hbm.at[wid], dst_v)
        plsc.subcore_barrier()

        @pl.loop(0, EROWS_W)
        def _(j):
            pltpu.sync_copy(ones_v, acc_sh.at[dst_v.at[j]], add=True)

        plsc.subcore_barrier()
        pltpu.sync_copy(
            acc_sh.at[pl.ds(sid * SUB_ROWS, SUB_ROWS)],
            out_hbm.at[cid].at[pl.ds(sid * SUB_ROWS, SUB_ROWS)])

    return k(dst3)


def _sc_edge_aggregate(xs, src3, dst3):
    @functools.partial(
        pl.kernel,
        out_type=jax.ShapeDtypeStruct((SC_CORES, N_PAD, F_DIM), jnp.float32),
        mesh=_sc_mesh(),
        scratch_types=[
            pltpu.VMEM((EROWS_W, ECHUNK), jnp.int32),
            pltpu.VMEM((EROWS_W, ECHUNK), jnp.int32),
            pltpu.VMEM((ECHUNK, F_DIM), jnp.float32),
            pltpu.VMEM_SHARED((N_PAD, F_DIM), jnp.float32),
        ],
    )
    def k(xs_hbm, src_hbm, dst_hbm, out_hbm, src_v, dst_v, rows_v, acc_sh):
        cid = lax.axis_index("c")
        sid = lax.axis_index("s")
        wid = sid * SC_CORES + cid

        _fill_const(rows_v, ECHUNK, F_DIM, 0.0)

        @pl.loop(0, SUB_ROWS // ECHUNK)
        def _(q):
            pltpu.sync_copy(
                rows_v,
                acc_sh.at[pl.ds(sid * SUB_ROWS + q * ECHUNK, ECHUNK)])

        pltpu.sync_copy(src_hbm.at[wid], src_v)
        pltpu.sync_copy(dst_hbm.at[wid], dst_v)
        plsc.subcore_barrier()

        @pl.loop(0, EROWS_W)
        def _(j):
            pltpu.sync_copy(xs_hbm.at[src_v.at[j]], rows_v)
            pltpu.sync_copy(rows_v, acc_sh.at[dst_v.at[j]], add=True)

        plsc.subcore_barrier()
        pltpu.sync_copy(
            acc_sh.at[pl.ds(sid * SUB_ROWS, SUB_ROWS)],
            out_hbm.at[cid].at[pl.ds(sid * SUB_ROWS, SUB_ROWS)])

    return k(xs, src3, dst3)


def _dot(a, b):
    return jax.lax.dot_general(
        a, b, (((1,), (0,)), ((), ())),
        precision=jax.lax.Precision.HIGHEST,
        preferred_element_type=jnp.float32)


def _k1_matmul_scale(x, W1, deg2):
    def body(x_ref, w_ref, d_ref, xs_ref, dinv_ref):
        deg = d_ref[0, :, 0:1] + d_ref[1, :, 0:1] + 1.0
        dinv = lax.rsqrt(deg)
        dinv_ref[...] = jnp.broadcast_to(dinv, (dinv.shape[0], 16))
        xs_ref[...] = _dot(x_ref[...], w_ref[...]) * dinv

    return pl.pallas_call(
        body,
        grid=(GRID,),
        in_specs=[
            pl.BlockSpec((BM, F_DIM), lambda i: (i, 0)),
            pl.BlockSpec((F_DIM, F_DIM), lambda i: (0, 0)),
            pl.BlockSpec((2, BM, F_DIM), lambda i: (0, i, 0)),
        ],
        out_specs=[
            pl.BlockSpec((BM, F_DIM), lambda i: (i, 0)),
            pl.BlockSpec((BM, 16), lambda i: (i, 0)),
        ],
        out_shape=[
            jax.ShapeDtypeStruct((N_PAD, F_DIM), jnp.float32),
            jax.ShapeDtypeStruct((N_NODES, 16), jnp.float32),
        ],
        compiler_params=pltpu.CompilerParams(
            dimension_semantics=("parallel",)),
    )(x, W1, deg2)


def _k2_combine_stats(parts, xs, dinv, b):
    def body(p_ref, xs_ref, dinv_ref, b_ref, h_ref, st_ref):
        h = (p_ref[0] + p_ref[1] + xs_ref[...]) * dinv_ref[..., 0:1] \
            + b_ref[...]
        h_ref[...] = h
        s1 = jnp.sum(h, axis=0, keepdims=True)
        s2 = jnp.sum(h * h, axis=0, keepdims=True)
        st_ref[...] = jnp.concatenate([s1, s2], axis=0)[None]

    return pl.pallas_call(
        body,
        grid=(GRID,),
        in_specs=[
            pl.BlockSpec((2, BM, F_DIM), lambda i: (0, i, 0)),
            pl.BlockSpec((BM, F_DIM), lambda i: (i, 0)),
            pl.BlockSpec((BM, 16), lambda i: (i, 0)),
            pl.BlockSpec((1, F_DIM), lambda i: (0, 0)),
        ],
        out_specs=[
            pl.BlockSpec((BM, F_DIM), lambda i: (i, 0)),
            pl.BlockSpec((1, 2, F_DIM), lambda i: (i, 0, 0)),
        ],
        out_shape=[
            jax.ShapeDtypeStruct((N_NODES, F_DIM), jnp.float32),
            jax.ShapeDtypeStruct((GRID, 2, F_DIM), jnp.float32),
        ],
        compiler_params=pltpu.CompilerParams(
            dimension_semantics=("parallel",)),
    )(parts, xs, dinv, b)


def _bn_coeffs(st, g, be):
    stats = jnp.sum(st, axis=0)
    m = stats[0:1] / N_NODES
    v = stats[1:2] / N_NODES - m * m
    a = g * lax.rsqrt(v + EPS)
    return m, a, be - m * a


def _k3_bn_relu_matmul_scale(h, st, g, be, W2, dinv):
    def body(h_ref, st_ref, g_ref, be_ref, w_ref, dinv_ref, xs_ref):
        _, a, c = _bn_coeffs(st_ref[...], g_ref[...], be_ref[...])
        hn = jnp.maximum(h_ref[...] * a + c, 0.0)
        xs_ref[...] = _dot(hn, w_ref[...]) * dinv_ref[..., 0:1]

    return pl.pallas_call(
        body,
        grid=(GRID,),
        in_specs=[
            pl.BlockSpec((BM, F_DIM), lambda i: (i, 0)),
            pl.BlockSpec((GRID, 2, F_DIM), lambda i: (0, 0, 0)),
            pl.BlockSpec((1, F_DIM), lambda i: (0, 0)),
            pl.BlockSpec((1, F_DIM), lambda i: (0, 0)),
            pl.BlockSpec((F_DIM, F_DIM), lambda i: (0, 0)),
            pl.BlockSpec((BM, 16), lambda i: (i, 0)),
        ],
        out_specs=pl.BlockSpec((BM, F_DIM), lambda i: (i, 0)),
        out_shape=jax.ShapeDtypeStruct((N_PAD, F_DIM), jnp.float32),
        compiler_params=pltpu.CompilerParams(
            dimension_semantics=("parallel",)),
    )(h, st, g, be, W2, dinv)


def _k5_classifier(h, st, g, be, x, Wc, bc):
    def body(h_ref, st_ref, g_ref, be_ref, x_ref, wc_ref, bc_ref, o_ref):
        _, a, c = _bn_coeffs(st_ref[...], g_ref[...], be_ref[...])
        hn = jnp.maximum(h_ref[...] * a + c, 0.0)
        z = (_dot(hn, wc_ref[0:F_DIM]) + _dot(x_ref[...], wc_ref[F_DIM:])
             + bc_ref[...])
        mx = jnp.max(z, axis=1, keepdims=True)
        e = jnp.exp(z - mx)
        lse = jnp.log(jnp.sum(e, axis=1, keepdims=True)) + mx
        o_ref[...] = z - lse

    return pl.pallas_call(
        body,
        grid=(GRID,),
        in_specs=[
            pl.BlockSpec((BM, F_DIM), lambda i: (i, 0)),
            pl.BlockSpec((GRID, 2, F_DIM), lambda i: (0, 0, 0)),
            pl.BlockSpec((1, F_DIM), lambda i: (0, 0)),
            pl.BlockSpec((1, F_DIM), lambda i: (0, 0)),
            pl.BlockSpec((BM, F_DIM), lambda i: (i, 0)),
            pl.BlockSpec((2 * F_DIM, OUT_DIM), lambda i: (0, 0)),
            pl.BlockSpec((1, OUT_DIM), lambda i: (0, 0)),
        ],
        out_specs=pl.BlockSpec((BM, OUT_DIM), lambda i: (i, 0)),
        out_shape=jax.ShapeDtypeStruct((N_NODES, OUT_DIM), jnp.float32),
        compiler_params=pltpu.CompilerParams(
            dimension_semantics=("parallel",)),
    )(h, st, g, be, x, Wc, bc)


def kernel(x, edge_index, W1, b1, g1, be1, W2, b2, g2, be2, Wc, bc):
    pad = jnp.full((2, E_PAD - N_EDGES), N_PAD - 1, edge_index.dtype)
    ei = jnp.concatenate([edge_index, pad], axis=1)
    src3 = ei[0].reshape(N_WORKERS, EROWS_W, ECHUNK)
    dst3 = ei[1].reshape(N_WORKERS, EROWS_W, ECHUNK)
    b1r = b1.reshape(1, F_DIM)
    g1r = g1.reshape(1, F_DIM)
    be1r = be1.reshape(1, F_DIM)
    b2r = b2.reshape(1, F_DIM)
    g2r = g2.reshape(1, F_DIM)
    be2r = be2.reshape(1, F_DIM)
    bcr = bc.reshape(1, OUT_DIM)

    deg2 = _sc_degree(dst3)
    xs1, dinv = _k1_matmul_scale(x, W1, deg2)
    p1 = _sc_edge_aggregate(xs1, src3, dst3)
    h1, st1 = _k2_combine_stats(p1, xs1, dinv, b1r)
    xs2 = _k3_bn_relu_matmul_scale(h1, st1, g1r, be1r, W2, dinv)
    p2 = _sc_edge_aggregate(xs2, src3, dst3)
    h2, st2 = _k2_combine_stats(p2, xs2, dinv, b2r)
    return _k5_classifier(h2, st2, g2r, be2r, x, Wc, bcr)


# speedup vs baseline: 9.3254x; 1.4053x over previous
import functools

import jax
import jax.numpy as jnp
from jax import lax
from jax.experimental import pallas as pl
from jax.experimental.pallas import tpu as pltpu
from jax.experimental.pallas import tpu_sc as plsc

N_NODES = 10000
N_PAD = 10240
N_EDGES = 320000
E_PAD = 327680
F_DIM = 128
OUT_DIM = 64
EPS = 1e-5

SC_CORES = 2
SC_SUBCORES = 16
N_WORKERS = SC_CORES * SC_SUBCORES

ECHUNK = 128
EROWS_W = 80
EROWS_TOT = N_WORKERS * EROWS_W
RA_ROWS = 112
RB_ROWS = 2 * EROWS_W - RA_ROWS
CORE0_ROWS = SC_SUBCORES * RA_ROWS
SUB_ROWS = N_PAD // SC_SUBCORES

BM = 1000
GRID = N_NODES // BM


@functools.cache
def _sc_mesh():
    return plsc.VectorSubcoreMesh(
        core_axis_name="c", subcore_axis_name="s",
        num_cores=SC_CORES, num_subcores=SC_SUBCORES)


def _fill_const(ref, nrows, ncols, value):
    @pl.loop(0, nrows)
    def _(r):
        @pl.loop(0, ncols // 16)
        def _(c):
            ref[r, pl.ds(c * 16, 16)] = jnp.full((16,), value, jnp.float32)


def _sc_degree(dst3):
    @functools.partial(
        pl.kernel,
        out_type=jax.ShapeDtypeStruct((SC_CORES, N_PAD, F_DIM), jnp.float32),
        mesh=_sc_mesh(),
        scratch_types=[
            pltpu.VMEM((EROWS_W, ECHUNK), jnp.int32),
            pltpu.VMEM((ECHUNK, F_DIM), jnp.float32),
            pltpu.VMEM_SHARED((N_PAD, F_DIM), jnp.float32),
        ],
    )
    def k(dst_hbm, out_hbm, dst_v, ones_v, acc_sh):
        cid = lax.axis_index("c")
        sid = lax.axis_index("s")
        wid = sid * SC_CORES + cid

        _fill_const(ones_v, ECHUNK, F_DIM, 0.0)

        @pl.loop(0, SUB_ROWS // ECHUNK)
        def _(q):
            pltpu.sync_copy(
                ones_v,
                acc_sh.at[pl.ds(sid * SUB_ROWS + q * ECHUNK, ECHUNK)])

        _fill_const(ones_v, ECHUNK, F_DIM, 1.0)
        pltpu.sync_copy(dst_hbm.at[pl.ds(wid * EROWS_W, EROWS_W)], dst_v)
        plsc.subcore_barrier()

        @pl.loop(0, EROWS_W)
        def _(j):
            pltpu.sync_copy(ones_v, acc_sh.at[dst_v.at[j]], add=True)

        plsc.subcore_barrier()
        pltpu.sync_copy(
            acc_sh.at[pl.ds(sid * SUB_ROWS, SUB_ROWS)],
            out_hbm.at[cid].at[pl.ds(sid * SUB_ROWS, SUB_ROWS)])

    return k(dst3)


SPAN = 8
NSPAN = EROWS_W // SPAN


def _sc_edge_aggregate(xs, src3, dst3):
    @functools.partial(
        pl.kernel,
        out_type=jax.ShapeDtypeStruct((SC_CORES, N_PAD, F_DIM), jnp.float32),
        mesh=_sc_mesh(),
        scratch_types=[
            pltpu.VMEM((SPAN, ECHUNK), jnp.int32),
            pltpu.VMEM((SPAN, ECHUNK), jnp.int32),
            pltpu.VMEM((SPAN, ECHUNK), jnp.int32),
            pltpu.VMEM((SPAN, ECHUNK), jnp.int32),
            pltpu.VMEM((ECHUNK, F_DIM), jnp.float32),
            pltpu.VMEM((ECHUNK, F_DIM), jnp.float32),
            pltpu.SemaphoreType.DMA,
            pltpu.VMEM_SHARED((N_PAD, F_DIM), jnp.float32),
        ],
    )
    def k(xs_hbm, src_hbm, dst_hbm, out_hbm, src_a, dst_a, src_b, dst_b,
          rows_a, rows_b, sem, acc_sh):
        cid = lax.axis_index("c")
        sid = lax.axis_index("s")
        nspan = jnp.where(cid == 0, RA_ROWS // SPAN, RB_ROWS // SPAN)
        nchunks = nspan * SPAN
        row_base = jnp.where(cid == 0, sid * RA_ROWS,
                             CORE0_ROWS + sid * RB_ROWS)

        def idx_slice(off):
            return pl.ds(pl.multiple_of(row_base + off, SPAN), SPAN)

        _fill_const(rows_a, ECHUNK, F_DIM, 0.0)

        @pl.loop(0, SUB_ROWS // ECHUNK)
        def _(q):
            pltpu.sync_copy(
                rows_a,
                acc_sh.at[pl.ds(sid * SUB_ROWS + q * ECHUNK, ECHUNK)])

        pltpu.sync_copy(src_hbm.at[idx_slice(0)], src_a)
        pltpu.sync_copy(dst_hbm.at[idx_slice(0)], dst_a)
        plsc.subcore_barrier()

        def wait_gather(buf):
            pltpu.make_async_copy(xs_hbm.at[pl.ds(0, ECHUNK)], buf,
                                  sem).wait()

        pltpu.async_copy(xs_hbm.at[src_a.at[0]], rows_a, sem)

        @pl.loop(0, nspan, step=2)
        def _(s):
            for p in (0, 1):
                sp = s + p
                src_c, dst_c = (src_a, dst_a) if p == 0 else (src_b, dst_b)
                src_n, dst_n = (src_b, dst_b) if p == 0 else (src_a, dst_a)
                for r in range(SPAN):
                    j = sp * SPAN + r
                    cur, nxt = (rows_a, rows_b) if r % 2 == 0 \
                        else (rows_b, rows_a)
                    wait_gather(cur)
                    if r < SPAN - 1:
                        pltpu.async_copy(
                            xs_hbm.at[src_c.at[r + 1]], nxt, sem)
                    else:
                        @pl.when(j + 1 < nchunks)
                        def _():
                            pltpu.async_copy(
                                xs_hbm.at[src_n.at[0]], nxt, sem)
                    if r == 1:
                        @pl.when(sp + 1 < nspan)
                        def _():
                            off = (sp + 1) * SPAN
                            pltpu.sync_copy(
                                src_hbm.at[idx_slice(off)], src_n)
                            pltpu.sync_copy(
                                dst_hbm.at[idx_slice(off)], dst_n)
                    pltpu.sync_copy(cur, acc_sh.at[dst_c.at[r]], add=True)

        plsc.subcore_barrier()
        pltpu.sync_copy(
            acc_sh.at[pl.ds(sid * SUB_ROWS, SUB_ROWS)],
            out_hbm.at[cid].at[pl.ds(sid * SUB_ROWS, SUB_ROWS)])

    return k(xs, src3, dst3)


def _dot(a, b):
    return jax.lax.dot_general(
        a, b, (((1,), (0,)), ((), ())),
        precision=jax.lax.Precision.HIGHEST,
        preferred_element_type=jnp.float32)


def _k1_matmul_scale(x, W1, deg2):
    def body(x_ref, w_ref, d_ref, xs_ref, dinv_ref):
        deg = d_ref[0, :, 0:1] + d_ref[1, :, 0:1] + 1.0
        dinv = lax.rsqrt(deg)
        dinv_ref[...] = jnp.broadcast_to(dinv, (dinv.shape[0], 16))
        xs_ref[...] = _dot(x_ref[...], w_ref[...]) * dinv

    return pl.pallas_call(
        body,
        grid=(GRID,),
        in_specs=[
            pl.BlockSpec((BM, F_DIM), lambda i: (i, 0)),
            pl.BlockSpec((F_DIM, F_DIM), lambda i: (0, 0)),
            pl.BlockSpec((2, BM, F_DIM), lambda i: (0, i, 0)),
        ],
        out_specs=[
            pl.BlockSpec((BM, F_DIM), lambda i: (i, 0)),
            pl.BlockSpec((BM, 16), lambda i: (i, 0)),
        ],
        out_shape=[
            jax.ShapeDtypeStruct((N_NODES, F_DIM), jnp.float32),
            jax.ShapeDtypeStruct((N_NODES, 16), jnp.float32),
        ],
        compiler_params=pltpu.CompilerParams(
            dimension_semantics=("parallel",)),
    )(x, W1, deg2)


def _k2_combine_stats(parts, xs, dinv, b):
    def body(p_ref, xs_ref, dinv_ref, b_ref, h_ref, st_ref):
        h = (p_ref[0] + p_ref[1] + xs_ref[...]) * dinv_ref[..., 0:1] \
            + b_ref[...]
        h_ref[...] = h
        s1 = jnp.sum(h, axis=0, keepdims=True)
        s2 = jnp.sum(h * h, axis=0, keepdims=True)
        st_ref[...] = jnp.concatenate([s1, s2], axis=0)[None]

    return pl.pallas_call(
        body,
        grid=(GRID,),
        in_specs=[
            pl.BlockSpec((2, BM, F_DIM), lambda i: (0, i, 0)),
            pl.BlockSpec((BM, F_DIM), lambda i: (i, 0)),
            pl.BlockSpec((BM, 16), lambda i: (i, 0)),
            pl.BlockSpec((1, F_DIM), lambda i: (0, 0)),
        ],
        out_specs=[
            pl.BlockSpec((BM, F_DIM), lambda i: (i, 0)),
            pl.BlockSpec((1, 2, F_DIM), lambda i: (i, 0, 0)),
        ],
        out_shape=[
            jax.ShapeDtypeStruct((N_NODES, F_DIM), jnp.float32),
            jax.ShapeDtypeStruct((GRID, 2, F_DIM), jnp.float32),
        ],
        compiler_params=pltpu.CompilerParams(
            dimension_semantics=("parallel",)),
    )(parts, xs, dinv, b)


def _bn_coeffs(st, g, be):
    stats = jnp.sum(st, axis=0)
    m = stats[0:1] / N_NODES
    v = stats[1:2] / N_NODES - m * m
    a = g * lax.rsqrt(v + EPS)
    return m, a, be - m * a


def _k3_bn_relu_matmul_scale(h, st, g, be, W2, dinv):
    def body(h_ref, st_ref, g_ref, be_ref, w_ref, dinv_ref, xs_ref):
        _, a, c = _bn_coeffs(st_ref[...], g_ref[...], be_ref[...])
        hn = jnp.maximum(h_ref[...] * a + c, 0.0)
        xs_ref[...] = _dot(hn, w_ref[...]) * dinv_ref[..., 0:1]

    return pl.pallas_call(
        body,
        grid=(GRID,),
        in_specs=[
            pl.BlockSpec((BM, F_DIM), lambda i: (i, 0)),
            pl.BlockSpec((GRID, 2, F_DIM), lambda i: (0, 0, 0)),
            pl.BlockSpec((1, F_DIM), lambda i: (0, 0)),
            pl.BlockSpec((1, F_DIM), lambda i: (0, 0)),
            pl.BlockSpec((F_DIM, F_DIM), lambda i: (0, 0)),
            pl.BlockSpec((BM, 16), lambda i: (i, 0)),
        ],
        out_specs=pl.BlockSpec((BM, F_DIM), lambda i: (i, 0)),
        out_shape=jax.ShapeDtypeStruct((N_NODES, F_DIM), jnp.float32),
        compiler_params=pltpu.CompilerParams(
            dimension_semantics=("parallel",)),
    )(h, st, g, be, W2, dinv)


def _k5_classifier(h, st, g, be, x, Wc, bc):
    def body(h_ref, st_ref, g_ref, be_ref, x_ref, wc_ref, bc_ref, o_ref):
        _, a, c = _bn_coeffs(st_ref[...], g_ref[...], be_ref[...])
        hn = jnp.maximum(h_ref[...] * a + c, 0.0)
        z = (_dot(hn, wc_ref[0:F_DIM]) + _dot(x_ref[...], wc_ref[F_DIM:])
             + bc_ref[...])
        mx = jnp.max(z, axis=1, keepdims=True)
        e = jnp.exp(z - mx)
        lse = jnp.log(jnp.sum(e, axis=1, keepdims=True)) + mx
        o_ref[...] = z - lse

    return pl.pallas_call(
        body,
        grid=(GRID,),
        in_specs=[
            pl.BlockSpec((BM, F_DIM), lambda i: (i, 0)),
            pl.BlockSpec((GRID, 2, F_DIM), lambda i: (0, 0, 0)),
            pl.BlockSpec((1, F_DIM), lambda i: (0, 0)),
            pl.BlockSpec((1, F_DIM), lambda i: (0, 0)),
            pl.BlockSpec((BM, F_DIM), lambda i: (i, 0)),
            pl.BlockSpec((2 * F_DIM, OUT_DIM), lambda i: (0, 0)),
            pl.BlockSpec((1, OUT_DIM), lambda i: (0, 0)),
        ],
        out_specs=pl.BlockSpec((BM, OUT_DIM), lambda i: (i, 0)),
        out_shape=jax.ShapeDtypeStruct((N_NODES, OUT_DIM), jnp.float32),
        compiler_params=pltpu.CompilerParams(
            dimension_semantics=("parallel",)),
    )(h, st, g, be, x, Wc, bc)


def kernel(x, edge_index, W1, b1, g1, be1, W2, b2, g2, be2, Wc, bc):
    pad_src = jnp.zeros((1, E_PAD - N_EDGES), edge_index.dtype)
    pad_dst = (N_NODES + jnp.arange(E_PAD - N_EDGES, dtype=edge_index.dtype)
               % (N_PAD - N_NODES))[None]
    ei = jnp.concatenate([edge_index, jnp.concatenate([pad_src, pad_dst])],
                         axis=1)
    src3 = ei[0].reshape(EROWS_TOT, ECHUNK)
    dst3 = ei[1].reshape(EROWS_TOT, ECHUNK)
    b1r = b1.reshape(1, F_DIM)
    g1r = g1.reshape(1, F_DIM)
    be1r = be1.reshape(1, F_DIM)
    b2r = b2.reshape(1, F_DIM)
    g2r = g2.reshape(1, F_DIM)
    be2r = be2.reshape(1, F_DIM)
    bcr = bc.reshape(1, OUT_DIM)

    deg2 = _sc_degree(dst3)
    xs1, dinv = _k1_matmul_scale(x, W1, deg2)
    p1 = _sc_edge_aggregate(xs1, src3, dst3)
    h1, st1 = _k2_combine_stats(p1, xs1, dinv, b1r)
    xs2 = _k3_bn_relu_matmul_scale(h1, st1, g1r, be1r, W2, dinv)
    p2 = _sc_edge_aggregate(xs2, src3, dst3)
    h2, st2 = _k2_combine_stats(p2, xs2, dinv, b2r)
    return _k5_classifier(h2, st2, g2r, be2r, x, Wc, bcr)
